# Initial kernel scaffold; baseline (speedup 1.0000x reference)
import jax
import jax.numpy as jnp
from jax.experimental import pallas as pl


def kernel(inputs, pos_table):
    raise NotImplementedError("write your pallas kernel here")


# speedup vs baseline: 1.6718x; 1.6718x over previous
import jax
import jax.numpy as jnp
from jax.experimental import pallas as pl

_BATCH = 4
_SEQ = 4096
_D = 1024
_BS = 512


def _add_kernel(x_ref, p_ref, o_ref):
    o_ref[...] = x_ref[...] + p_ref[...]


def kernel(inputs, pos_table):
    return pl.pallas_call(
        _add_kernel,
        grid=(_SEQ // _BS, _BATCH),
        in_specs=[
            pl.BlockSpec((1, _BS, _D), lambda s, b: (b, s, 0)),
            pl.BlockSpec((_BS, _D), lambda s, b: (s, 0)),
        ],
        out_specs=pl.BlockSpec((1, _BS, _D), lambda s, b: (b, s, 0)),
        out_shape=jax.ShapeDtypeStruct(inputs.shape, inputs.dtype),
    )(inputs, pos_table)


# speedup vs baseline: 1.9467x; 1.1644x over previous
import jax
import jax.numpy as jnp
from jax.experimental import pallas as pl

_BATCH = 4
_SEQ = 4096
_D = 1024
_BS = 512


def _add_kernel(x_ref, p_ref, o_ref):
    o_ref[...] = x_ref[...] + p_ref[...][None]


def kernel(inputs, pos_table):
    return pl.pallas_call(
        _add_kernel,
        grid=(_SEQ // _BS,),
        in_specs=[
            pl.BlockSpec((_BATCH, _BS, _D), lambda s: (0, s, 0)),
            pl.BlockSpec((_BS, _D), lambda s: (s, 0)),
        ],
        out_specs=pl.BlockSpec((_BATCH, _BS, _D), lambda s: (0, s, 0)),
        out_shape=jax.ShapeDtypeStruct(inputs.shape, inputs.dtype),
    )(inputs, pos_table)
